# baseline (speedup 1.0000x reference)
import functools
import math

import jax
import jax.numpy as jnp
from jax import lax
from jax.experimental import pallas as pl
from jax.experimental.pallas import tpu as pltpu
from jax.experimental.pallas import tpu_sc as plsc

_LANES = 16
_NBUF = 4
_SUB = 8
_LANE_TILE = 128


def _make_sc_gather(b: int, s: int, v2: int, d: int, scale: float,
                    num_cores: int, num_subcores: int):
    nw = num_cores * num_subcores
    bpw = b // nw
    d2 = 2 * d
    fh = d // _SUB
    groups = s // _NBUF
    nblk = bpw // _LANES
    mesh = plsc.VectorSubcoreMesh(core_axis_name="c", subcore_axis_name="s")

    @functools.partial(
        pl.kernel,
        out_type=jax.ShapeDtypeStruct((s, fh, nw, _SUB, bpw), jnp.float32),
        mesh=mesh,
        scratch_types=[
            pltpu.VMEM((s, bpw), jnp.int32),
            pltpu.VMEM((_NBUF, bpw, d), jnp.float32),
            pltpu.VMEM((_NBUF, fh, _SUB, bpw + 1), jnp.float32),
            pltpu.SemaphoreType.DMA((_NBUF,)),
            pltpu.SemaphoreType.DMA((_NBUF,)),
        ],
        compiler_params=pltpu.CompilerParams(use_tc_tiling_on_sc=False,
                                             needs_layout_passes=False),
    )
    def sc_gather(tok_hbm, table_hbm, out_hbm, idx_v, rows_v, trans_v,
                  gsem, ssem):
        wid = lax.axis_index("s") * num_cores + lax.axis_index("c")
        b0 = wid * bpw
        iota = lax.iota(jnp.int32, _LANES)
        pltpu.sync_copy(tok_hbm.at[pl.ds(0, s), pl.ds(b0, bpw)], idx_v)

        def issue_gather(step, buf):
            pltpu.async_copy(
                table_hbm.at[idx_v.at[step]], rows_v.at[buf], gsem.at[buf]
            )

        def wait_gather(buf):
            pltpu.make_async_copy(
                table_hbm.at[pl.ds(0, bpw)], rows_v.at[buf], gsem.at[buf]
            ).wait()

        def out_slab(step):
            return out_hbm.at[step, pl.ds(0, fh), wid, pl.ds(0, _SUB),
                              pl.ds(0, bpw)]

        def trans_src(buf):
            return trans_v.at[buf, pl.ds(0, fh), pl.ds(0, _SUB),
                              pl.ds(0, bpw)]

        def wait_store(buf):
            pltpu.make_async_copy(
                trans_src(buf), out_slab(0), ssem.at[buf]
            ).wait()

        for buf in range(_NBUF):
            issue_gather(buf, buf)

        def group_body(g, carry):
            i0 = g * _NBUF
            for buf in range(_NBUF):
                i = i0 + buf
                bprev = (buf - 1) % _NBUF

                @pl.when(jnp.logical_and(i >= 1, i + _NBUF - 1 < s))
                def _():
                    issue_gather(i + _NBUF - 1, bprev)

                wait_gather(buf)

                @pl.when(i >= _NBUF)
                def _():
                    wait_store(buf)

                fidx = [(((iota + f0) // _SUB), ((iota + f0) % _SUB))
                        for f0 in range(0, d, _LANES)]

                @plsc.parallel_loop(0, bpw, unroll=4)
                def _(t):
                    idx_t = iota * 0 + t
                    for k, f0 in enumerate(range(0, d, _LANES)):
                        x = rows_v[buf, t, pl.ds(f0, _LANES)] * scale
                        plsc.store_scatter(trans_v.at[buf],
                                           [fidx[k][0], fidx[k][1], idx_t],
                                           x)

                pltpu.async_copy(trans_src(buf), out_slab(i), ssem.at[buf])
            return carry

        lax.fori_loop(0, groups, group_body, 0)
        for buf in range(_NBUF):
            wait_store(buf)

    return sc_gather


def _make_sc_repack(v: int, d: int, num_cores: int, num_subcores: int):
    nw = num_cores * num_subcores
    d2 = 2 * d
    groups = v // _SUB
    gpw = groups // nw
    rem = groups - gpw * nw
    cgrp = 18
    nchunk = gpw // cgrp
    crow = cgrp * _SUB
    mesh = plsc.VectorSubcoreMesh(core_axis_name="c", subcore_axis_name="s")

    @functools.partial(
        pl.kernel,
        out_type=jax.ShapeDtypeStruct((v // 2, d2), jnp.float32),
        mesh=mesh,
        scratch_types=[
            pltpu.VMEM((2, crow, d), jnp.float32),
            pltpu.VMEM((2, crow // 2, d2), jnp.float32),
            pltpu.SemaphoreType.DMA((2,)),
            pltpu.SemaphoreType.DMA((2,)),
        ],
        compiler_params=pltpu.CompilerParams(use_tc_tiling_on_sc=True),
    )
    def sc_repack(tab_hbm, out_hbm, a_v, b_v, gsem, ssem):
        wid = lax.axis_index("s") * num_cores + lax.axis_index("c")
        row0 = wid * gpw * _SUB

        def issue_load(c, buf):
            off = pl.multiple_of(row0 + c * crow, _SUB)
            pltpu.async_copy(
                tab_hbm.at[pl.ds(off, crow), pl.ds(0, d)],
                a_v.at[buf], gsem.at[buf])

        def wait_load(buf):
            pltpu.make_async_copy(
                tab_hbm.at[pl.ds(0, crow), pl.ds(0, d)], a_v.at[buf],
                gsem.at[buf]).wait()

        def wait_store(buf):
            pltpu.make_async_copy(
                b_v.at[buf], out_hbm.at[pl.ds(0, crow // 2)],
                ssem.at[buf]).wait()

        issue_load(0, 0)
        issue_load(1, 1)

        def chunk_body(c, carry):
            buf = lax.rem(c, 2)
            wait_load(buf)

            @plsc.parallel_loop(0, crow // 2, unroll=2)
            def _(r):
                for k in range(d2 // _LANES):
                    b_v[buf, r, pl.ds(k * _LANES, _LANES)] = a_v[
                        buf, 2 * r + (k * _LANES) // d,
                        pl.ds((k * _LANES) % d, _LANES)]

            @pl.when(c >= 2)
            def _():
                wait_store(buf)

            ooff = pl.multiple_of((row0 + c * crow) // 2, _SUB)
            pltpu.async_copy(
                b_v.at[buf],
                out_hbm.at[pl.ds(ooff, crow // 2)],
                ssem.at[buf])

            @pl.when(c + 2 < nchunk)
            def _():
                issue_load(c + 2, buf)
            return carry

        lax.fori_loop(0, nchunk, chunk_body, 0)
        wait_store(0)
        wait_store(1)

        @pl.when(wid == nw - 1)
        def _():
            r0 = groups * _SUB - rem * _SUB
            pltpu.sync_copy(
                tab_hbm.at[pl.ds(r0, rem * _SUB), pl.ds(0, d)],
                a_v.at[0, pl.ds(0, rem * _SUB)])

            @plsc.parallel_loop(0, rem * _SUB // 2)
            def _(r):
                for k in range(d2 // _LANES):
                    b_v[0, r, pl.ds(k * _LANES, _LANES)] = a_v[
                        0, 2 * r + (k * _LANES) // d,
                        pl.ds((k * _LANES) % d, _LANES)]

            pltpu.sync_copy(b_v.at[0, pl.ds(0, rem * _SUB // 2)],
                            out_hbm.at[pl.ds(r0 // 2, rem * _SUB // 2)])

    return sc_repack


def kernel(tokens, table):
    b, s = tokens.shape
    v, d = table.shape
    info = plsc.get_sparse_core_info()
    nw = info.num_cores * info.num_subcores
    assert b // nw == _LANE_TILE
    assert d % _LANES == 0 and s % _NBUF == 0 and d % _SUB == 0
    tok_t = jnp.swapaxes(tokens, 0, 1).astype(jnp.int32)
    table_rm = _make_sc_repack(v, d, info.num_cores,
                               info.num_subcores)(table).reshape(v, d)
    out5 = _make_sc_gather(b, s, v, d, math.sqrt(d), info.num_cores,
                           info.num_subcores)(tok_t, table_rm)
    return out5.transpose(2, 4, 0, 1, 3).reshape(b, s, d)


# speedup vs baseline: 1.0139x; 1.0139x over previous
import functools
import math

import jax
import jax.numpy as jnp
from jax import lax
from jax.experimental import pallas as pl
from jax.experimental.pallas import tpu as pltpu
from jax.experimental.pallas import tpu_sc as plsc

_LANES = 16
_NBUF = 4
_SUB = 8
_LANE_TILE = 128


def _make_sc_gather(b: int, s: int, v2: int, d: int, scale: float,
                    num_cores: int, num_subcores: int):
    nw = num_cores * num_subcores
    bpw = b // nw
    d2 = 2 * d
    fh = d // _SUB
    groups = s // _NBUF
    nblk = bpw // _LANES
    mesh = plsc.VectorSubcoreMesh(core_axis_name="c", subcore_axis_name="s")

    @functools.partial(
        pl.kernel,
        out_type=jax.ShapeDtypeStruct((s, fh, nw, _SUB, bpw), jnp.float32),
        mesh=mesh,
        scratch_types=[
            pltpu.VMEM((s, bpw), jnp.int32),
            pltpu.VMEM((_NBUF, bpw, d), jnp.float32),
            pltpu.VMEM((_NBUF, fh, _SUB, bpw + 1), jnp.float32),
            pltpu.SemaphoreType.DMA((_NBUF,)),
            pltpu.SemaphoreType.DMA((_NBUF,)),
        ],
        compiler_params=pltpu.CompilerParams(use_tc_tiling_on_sc=False,
                                             needs_layout_passes=False),
    )
    def sc_gather(tok_hbm, table_hbm, out_hbm, idx_v, rows_v, trans_v,
                  gsem, ssem):
        wid = lax.axis_index("s") * num_cores + lax.axis_index("c")
        b0 = wid * bpw
        iota = lax.iota(jnp.int32, _LANES)
        pltpu.sync_copy(tok_hbm.at[pl.ds(0, s), pl.ds(b0, bpw)], idx_v)

        def issue_gather(step, buf):
            pltpu.async_copy(
                table_hbm.at[idx_v.at[step]], rows_v.at[buf], gsem.at[buf]
            )

        def wait_gather(buf):
            pltpu.make_async_copy(
                table_hbm.at[pl.ds(0, bpw)], rows_v.at[buf], gsem.at[buf]
            ).wait()

        def out_slab(step):
            return out_hbm.at[step, pl.ds(0, fh), wid, pl.ds(0, _SUB),
                              pl.ds(0, bpw)]

        def trans_src(buf):
            return trans_v.at[buf, pl.ds(0, fh), pl.ds(0, _SUB),
                              pl.ds(0, bpw)]

        def wait_store(buf):
            pltpu.make_async_copy(
                trans_src(buf), out_slab(0), ssem.at[buf]
            ).wait()

        for buf in range(_NBUF):
            issue_gather(buf, buf)

        def group_body(g, carry):
            i0 = g * _NBUF
            for buf in range(_NBUF):
                i = i0 + buf
                bprev = (buf - 1) % _NBUF

                @pl.when(jnp.logical_and(i >= 1, i + _NBUF - 1 < s))
                def _():
                    issue_gather(i + _NBUF - 1, bprev)

                wait_gather(buf)

                @pl.when(i >= _NBUF)
                def _():
                    wait_store(buf)

                fidx = [(((iota + f0) // _SUB), ((iota + f0) % _SUB))
                        for f0 in range(0, d, _LANES)]

                @plsc.parallel_loop(0, bpw, unroll=4)
                def _(t):
                    idx_t = iota * 0 + t
                    for k, f0 in enumerate(range(0, d, _LANES)):
                        x = rows_v[buf, t, pl.ds(f0, _LANES)] * scale
                        plsc.store_scatter(trans_v.at[buf],
                                           [fidx[k][0], fidx[k][1], idx_t],
                                           x)

                pltpu.async_copy(trans_src(buf), out_slab(i), ssem.at[buf])
            return carry

        lax.fori_loop(0, groups, group_body, 0)
        for buf in range(_NBUF):
            wait_store(buf)

    return sc_gather


def kernel(tokens, table):
    b, s = tokens.shape
    v, d = table.shape
    info = plsc.get_sparse_core_info()
    nw = info.num_cores * info.num_subcores
    assert b // nw == _LANE_TILE
    assert d % _LANES == 0 and s % _NBUF == 0 and d % _SUB == 0
    tok_t = jnp.swapaxes(tokens, 0, 1).astype(jnp.int32)
    out5 = _make_sc_gather(b, s, v, d, math.sqrt(d), info.num_cores,
                           info.num_subcores)(tok_t, table)
    return out5.transpose(2, 4, 0, 1, 3).reshape(b, s, d)
